# baseline (speedup 1.0000x reference)
import jax
import jax.numpy as jnp
from jax.experimental import pallas as pl
from jax.experimental.pallas import tpu as pltpu

_NUM_ARCHS = 8
_R, _C = 2048, 4096
_NCH = 16
_CH = _R // _NCH


def _dma_copy_kernel(id_ref, w_ref, o_ref, buf, rsem, wsem):
    a = id_ref[0]
    for i in range(_NCH):
        pltpu.make_async_copy(
            w_ref.at[a, pl.ds(i * _CH, _CH), :], buf.at[i], rsem.at[i]
        ).start()
    for i in range(_NCH):
        pltpu.make_async_copy(
            w_ref.at[a, pl.ds(i * _CH, _CH), :], buf.at[i], rsem.at[i]
        ).wait()
        pltpu.make_async_copy(
            buf.at[i], o_ref.at[pl.ds(i * _CH, _CH), :], wsem.at[i]
        ).start()
    for i in range(_NCH):
        pltpu.make_async_copy(
            buf.at[i], o_ref.at[pl.ds(i * _CH, _CH), :], wsem.at[i]
        ).wait()


def kernel(W, arch_id):
    idx = jnp.asarray(arch_id, jnp.int32).reshape((1,))
    return pl.pallas_call(
        _dma_copy_kernel,
        grid_spec=pltpu.PrefetchScalarGridSpec(
            num_scalar_prefetch=1,
            grid=(1,),
            in_specs=[pl.BlockSpec(memory_space=pl.ANY)],
            out_specs=pl.BlockSpec(memory_space=pl.ANY),
            scratch_shapes=[
                pltpu.VMEM((_NCH, _CH, _C), jnp.float32),
                pltpu.SemaphoreType.DMA((_NCH,)),
                pltpu.SemaphoreType.DMA((_NCH,)),
            ],
        ),
        out_shape=jax.ShapeDtypeStruct((_R, _C), W.dtype),
    )(idx, W)


# speedup vs baseline: 1.0245x; 1.0245x over previous
import jax
import jax.numpy as jnp
from jax.experimental import pallas as pl
from jax.experimental.pallas import tpu as pltpu

_NUM_ARCHS = 8
_R, _C = 2048, 4096
_NCH = 4
_CH = _R // _NCH


def _dma_copy_kernel(id_ref, w_ref, o_ref, buf, rsem, wsem):
    a = id_ref[0]
    for i in range(_NCH):
        pltpu.make_async_copy(
            w_ref.at[a, pl.ds(i * _CH, _CH), :], buf.at[i], rsem.at[i]
        ).start()
    for i in range(_NCH):
        pltpu.make_async_copy(
            w_ref.at[a, pl.ds(i * _CH, _CH), :], buf.at[i], rsem.at[i]
        ).wait()
        pltpu.make_async_copy(
            buf.at[i], o_ref.at[pl.ds(i * _CH, _CH), :], wsem.at[i]
        ).start()
    for i in range(_NCH):
        pltpu.make_async_copy(
            buf.at[i], o_ref.at[pl.ds(i * _CH, _CH), :], wsem.at[i]
        ).wait()


def kernel(W, arch_id):
    idx = jnp.asarray(arch_id, jnp.int32).reshape((1,))
    return pl.pallas_call(
        _dma_copy_kernel,
        grid_spec=pltpu.PrefetchScalarGridSpec(
            num_scalar_prefetch=1,
            grid=(1,),
            in_specs=[pl.BlockSpec(memory_space=pl.ANY)],
            out_specs=pl.BlockSpec(memory_space=pl.ANY),
            scratch_shapes=[
                pltpu.VMEM((_NCH, _CH, _C), jnp.float32),
                pltpu.SemaphoreType.DMA((_NCH,)),
                pltpu.SemaphoreType.DMA((_NCH,)),
            ],
        ),
        out_shape=jax.ShapeDtypeStruct((_R, _C), W.dtype),
    )(idx, W)


# speedup vs baseline: 1.0357x; 1.0110x over previous
import jax
import jax.numpy as jnp
from jax.experimental import pallas as pl
from jax.experimental.pallas import tpu as pltpu

_NUM_ARCHS = 8
_R, _C = 2048, 4096
_NCH = 2
_CH = _R // _NCH


def _dma_copy_kernel(id_ref, w_ref, o_ref, buf, rsem, wsem):
    a = id_ref[0]
    for i in range(_NCH):
        pltpu.make_async_copy(
            w_ref.at[a, pl.ds(i * _CH, _CH), :], buf.at[i], rsem.at[i]
        ).start()
    for i in range(_NCH):
        pltpu.make_async_copy(
            w_ref.at[a, pl.ds(i * _CH, _CH), :], buf.at[i], rsem.at[i]
        ).wait()
        pltpu.make_async_copy(
            buf.at[i], o_ref.at[pl.ds(i * _CH, _CH), :], wsem.at[i]
        ).start()
    for i in range(_NCH):
        pltpu.make_async_copy(
            buf.at[i], o_ref.at[pl.ds(i * _CH, _CH), :], wsem.at[i]
        ).wait()


def kernel(W, arch_id):
    idx = jnp.asarray(arch_id, jnp.int32).reshape((1,))
    return pl.pallas_call(
        _dma_copy_kernel,
        grid_spec=pltpu.PrefetchScalarGridSpec(
            num_scalar_prefetch=1,
            grid=(1,),
            in_specs=[pl.BlockSpec(memory_space=pl.ANY)],
            out_specs=pl.BlockSpec(memory_space=pl.ANY),
            scratch_shapes=[
                pltpu.VMEM((_NCH, _CH, _C), jnp.float32),
                pltpu.SemaphoreType.DMA((_NCH,)),
                pltpu.SemaphoreType.DMA((_NCH,)),
            ],
        ),
        out_shape=jax.ShapeDtypeStruct((_R, _C), W.dtype),
    )(idx, W)
